# Initial kernel scaffold; baseline (speedup 1.0000x reference)
import jax
import jax.numpy as jnp
from jax.experimental import pallas as pl


def kernel(x, params, kernals):
    raise NotImplementedError("write your pallas kernel here")


# speedup vs baseline: 4.4095x; 4.4095x over previous
import jax
import jax.numpy as jnp
from jax.experimental import pallas as pl
from jax.experimental.pallas import tpu as pltpu

_BIG = 1e30
_RS2 = 0.7071067811865476
_KS = 32


def _gelu(v):
    return v * 0.5 * (1.0 + jax.lax.erf(v * _RS2))


def _bf(x):
    return x.astype(jnp.bfloat16)


def _dot(a, b):
    return jnp.dot(_bf(a), _bf(b), preferred_element_type=jnp.float32)


def _hilo_rows(t):
    hi = t.astype(jnp.bfloat16)
    lo = (t - hi.astype(jnp.float32)).astype(jnp.bfloat16)
    return jnp.concatenate([hi, lo], axis=0)


def _hilo_cols(t):
    hi = t.astype(jnp.bfloat16)
    lo = (t - hi.astype(jnp.float32)).astype(jnp.bfloat16)
    return jnp.concatenate([hi, lo], axis=1)


def _full_spec(shape):
    nd = len(shape)
    return pl.BlockSpec(shape, lambda b, _n=nd: (0,) * _n)


def _sample_spec(shape):
    nd = len(shape)
    return pl.BlockSpec((1,) + shape[1:], lambda b, _n=nd: (b,) + (0,) * (_n - 1))


def _paiconv_body(K, C, CP, x_ref, xn_ref, f_ref, mlpw_ref, mlpb_ref,
                  convw_ref, cvec_ref, mow_ref, kt_ref, o_ref):
    pts = x_ref[0]
    ptsn = xn_ref[0]
    feat = f_ref[0]
    N = pts.shape[1]
    CT = 3 + C

    g = _dot(ptsn, pts)
    sqr = jnp.sum(pts * pts, axis=0, keepdims=True)
    sqc = jnp.sum(ptsn * ptsn, axis=1, keepdims=True)
    d2 = (sqr - 2.0 * g) + sqc
    iota = jax.lax.broadcasted_iota(jnp.int32, (N, N), 0)
    tcat = _hilo_rows(jnp.concatenate([pts, feat], axis=0))

    work = d2
    perms = []
    feats = []
    xn0 = None
    for k in range(K):
        mn = jnp.min(work, axis=0, keepdims=True)
        sel = work <= mn
        idxk = jnp.min(jnp.where(sel, iota, jnp.int32(N)), axis=0,
                       keepdims=True)
        oh = iota == idxk
        work = jnp.where(oh, jnp.float32(_BIG), work)
        g2 = jnp.dot(tcat, oh.astype(jnp.bfloat16),
                     preferred_element_type=jnp.float32)
        gk = g2[:CT] + g2[CT:]
        xn = gk[:3]
        fe = gk[3:]
        if k == 0:
            xn0 = xn
        xrel = xn - xn0
        sq = jnp.sum(xrel * xrel, axis=0, keepdims=True)
        dis = jnp.sqrt(jnp.maximum(sq, 1e-12))
        pk = jnp.dot(kt_ref[...], _bf(xrel),
                     preferred_element_type=jnp.float32)
        if k == 0:
            m0 = (jax.lax.broadcasted_iota(jnp.int32, (_KS, 1), 0) == 0)
            pk = pk + m0.astype(jnp.float32)
        pk = jnp.maximum(pk, 0.0)
        x7 = jnp.concatenate([xn0, xrel, dis], axis=0)
        xf = jnp.dot(mlpw_ref[...], _bf(x7),
                     preferred_element_type=jnp.float32) + mlpb_ref[...]
        fcat = jnp.concatenate([fe, xf], axis=0)
        if CP > 2 * C:
            fcat = jnp.concatenate(
                [fcat, jnp.zeros((CP - 2 * C, N), jnp.float32)], axis=0)
        perms.append(pk)
        feats.append(_bf(fcat))

    s1 = perms[0]
    for pk in perms[1:]:
        s1 = s1 + pk
    inv1 = 1.0 / (s1 + 1e-6)
    qs = []
    s2 = None
    for pk in perms:
        q = pk * inv1
        q = q * q
        qs.append(q)
        s2 = q if s2 is None else s2 + q
    inv2 = 1.0 / (s2 + 1e-6)

    acc = jnp.zeros((_KS, CP, N), jnp.float32)
    for k in range(K):
        w = qs[k] * inv2
        w = jnp.where(w > 0.1, w, 0.0)
        wb = _bf(w).astype(jnp.float32)
        acc = acc + feats[k][None].astype(jnp.float32) * wb[:, None, :]
    out1 = acc.reshape(_KS * CP, N)

    res = jnp.dot(convw_ref[...], _bf(out1),
                  preferred_element_type=jnp.float32)
    res = res + jnp.dot(mow_ref[...], _bf(feat),
                        preferred_element_type=jnp.float32)
    res = (res + cvec_ref[..., 0:1]) * cvec_ref[..., 1:2] + cvec_ref[..., 2:3]
    o_ref[0] = _gelu(res)


def _paiconv(x, xn, feature, pr, kt, K):
    B, _, N = x.shape
    C = feature.shape[1]
    out = pr['conv_W'].shape[0]
    CP = max(2 * C, 8)
    w = pr['conv_W'].reshape(out, 2 * C, _KS).transpose(0, 2, 1)
    if CP > 2 * C:
        w = jnp.pad(w, ((0, 0), (0, 0), (0, CP - 2 * C)))
    convw = _bf(w.reshape(out, _KS * CP))
    s = 1.0 / jnp.sqrt(jnp.float32(1.0 + 1e-5))
    cvec = jnp.stack([pr['conv_b'] + pr['mlpout_b'],
                      pr['bn_g'] * s, pr['bn_b']], axis=1)
    mlpb = pr['mlp_b'][:, None]
    mow = _bf(pr['mlpout_W'])
    mlpw = _bf(pr['mlp_W'])
    ktb = _bf(kt)

    body = lambda *refs: _paiconv_body(K, C, CP, *refs)
    return pl.pallas_call(
        body,
        grid=(B,),
        in_specs=[
            _sample_spec(x.shape),
            _sample_spec(xn.shape),
            _sample_spec(feature.shape),
            _full_spec(mlpw.shape),
            _full_spec(mlpb.shape),
            _full_spec(convw.shape),
            _full_spec(cvec.shape),
            _full_spec(mow.shape),
            _full_spec(ktb.shape),
        ],
        out_specs=_sample_spec((B, out, N)),
        out_shape=jax.ShapeDtypeStruct((B, out, N), jnp.float32),
        compiler_params=pltpu.CompilerParams(
            dimension_semantics=("arbitrary",)),
    )(x, xn, feature, mlpw, mlpb, convw, cvec, mow, ktb)


def _pool_body(P, x_ref, xn_ref, fn_ref, o_ref):
    pts = x_ref[0]
    ptsn = xn_ref[0]
    featn = fn_ref[0]
    C = featn.shape[1]
    N = pts.shape[1]
    sub = ptsn[:P]
    g = _dot(sub, pts)
    sqa = jnp.sum(pts * pts, axis=0, keepdims=True)
    sqq = jnp.sum(sub * sub, axis=1, keepdims=True)
    d2 = (sqq - 2.0 * g) + sqa
    iota = jax.lax.broadcasted_iota(jnp.int32, (P, N), 1)
    tcat = _hilo_cols(featn)
    work = d2
    pooled = jnp.full((P, C), -_BIG, jnp.float32)
    for _ in range(20):
        mn = jnp.min(work, axis=1, keepdims=True)
        sel = work <= mn
        idx = jnp.min(jnp.where(sel, iota, jnp.int32(N)), axis=1,
                      keepdims=True)
        oh = iota == idx
        work = jnp.where(oh, jnp.float32(_BIG), work)
        g1 = jnp.dot(oh.astype(jnp.bfloat16), tcat,
                     preferred_element_type=jnp.float32)
        pooled = jnp.maximum(pooled, g1[:, :C] + g1[:, C:])
    o_ref[0] = pooled


def _pool(x, xn, fn, P):
    B, _, N = x.shape
    C = fn.shape[2]
    body = lambda *refs: _pool_body(P, *refs)
    return pl.pallas_call(
        body,
        grid=(B,),
        in_specs=[
            _sample_spec(x.shape),
            _sample_spec(xn.shape),
            _sample_spec(fn.shape),
        ],
        out_specs=_sample_spec((B, P, C)),
        out_shape=jax.ShapeDtypeStruct((B, P, C), jnp.float32),
        compiler_params=pltpu.CompilerParams(
            dimension_semantics=("arbitrary",)),
    )(x, xn, fn)


def _head1_body(h_ref, w5t_ref, g_ref, b_ref, o_ref):
    hb = h_ref[0]
    c = _dot(hb, w5t_ref[...])
    c = _gelu(c * g_ref[...] + b_ref[...])
    h1 = jnp.max(c, axis=0, keepdims=True)
    h2 = jnp.mean(c, axis=0, keepdims=True)
    o_ref[0] = jnp.concatenate([h1, h2], axis=1)


def _head1(ht, w5t, g5, b5):
    B = ht.shape[0]
    E = w5t.shape[1]
    return pl.pallas_call(
        _head1_body,
        grid=(B,),
        in_specs=[
            _sample_spec(ht.shape),
            _full_spec(w5t.shape),
            _full_spec(g5.shape),
            _full_spec(b5.shape),
        ],
        out_specs=_sample_spec((B, 1, 2 * E)),
        out_shape=jax.ShapeDtypeStruct((B, 1, 2 * E), jnp.float32),
        compiler_params=pltpu.CompilerParams(
            dimension_semantics=("arbitrary",)),
    )(ht, w5t, g5, b5)


def _head2_body(h_ref, w1t_ref, g6_ref, b6_ref, w2t_ref, c2_ref, g7_ref,
                b7_ref, w3t_ref, c3_ref, o_ref):
    h = h_ref[...]
    a = _gelu(_dot(h, w1t_ref[...]) * g6_ref[...] + b6_ref[...])
    a = _gelu((_dot(a, w2t_ref[...]) + c2_ref[...]) * g7_ref[...]
              + b7_ref[...])
    o_ref[...] = _dot(a, w3t_ref[...]) + c3_ref[...]


def kernel(x, params, kernals):
    B, _, N = x.shape
    p = params
    kt = kernals.T

    def conv(xc, feat, pr, K):
        xnc = jnp.transpose(xc, (0, 2, 1))
        return _paiconv(xc, xnc, feat, pr, kt, K)

    def pool(xc, feat, P):
        xnc = jnp.transpose(xc, (0, 2, 1))
        fn = jnp.transpose(feat, (0, 2, 1))
        pooled = _pool(xc, xnc, fn, P)
        return xc[:, :, :P], jnp.transpose(pooled, (0, 2, 1))

    feature = conv(x, x, p['c1'], 20)
    x, feature = pool(x, feature, N // 4)
    _, x1 = pool(x, feature, N // 32)
    feature = conv(x, feature, p['c2'], 20)
    x, feature = pool(x, feature, N // 8)
    _, x2 = pool(x, feature, N // 32)
    feature = conv(x, feature, p['c3'], 20)
    x, feature = pool(x, feature, N // 16)
    _, x3 = pool(x, feature, N // 32)
    feature = conv(x, feature, p['c4'], 10)
    _, feature = pool(x, feature, N // 32)

    h = jnp.concatenate([x1, x2, x3, feature], axis=1)
    ht = jnp.transpose(h, (0, 2, 1))
    s = 1.0 / jnp.sqrt(jnp.float32(1.0 + 1e-5))
    hh = _head1(ht, p['conv5_W'].T, (p['bn5_g'] * s)[None, :],
                p['bn5_b'][None, :])[:, 0, :]

    out_ch = p['lin3_W'].shape[0]
    return pl.pallas_call(
        _head2_body,
        out_shape=jax.ShapeDtypeStruct((B, out_ch), jnp.float32),
    )(hh, p['lin1_W'].T, (p['bn6_g'] * s)[None, :], p['bn6_b'][None, :],
      p['lin2_W'].T, p['lin2_b'][None, :], (p['bn7_g'] * s)[None, :],
      p['bn7_b'][None, :], p['lin3_W'].T, p['lin3_b'][None, :])
